# Initial kernel scaffold; baseline (speedup 1.0000x reference)
import jax
import jax.numpy as jnp
from jax.experimental import pallas as pl


def kernel(x, edge_index, batch, W1, b1, W2, b2, W3, b3, Wl, bl):
    raise NotImplementedError("write your pallas kernel here")


# speedup vs baseline: 9.8081x; 9.8081x over previous
import functools

import jax
import jax.numpy as jnp
from jax import lax
from jax.experimental import pallas as pl
from jax.experimental.pallas import tpu as pltpu
from jax.experimental.pallas import tpu_sc as plsc

N = 10000
NP = 10240
D = 128
NG = 64
E = 320000
NC = 2
NS = 16
NW = NC * NS
CHUNK = 128
CPW = 79
EP = NW * CPW * CHUNK
RPT = NP // NS


def _zero_rows(buf, nrows, ncols):
    def body(i, _):
        r = i // (ncols // 16)
        c = (i % (ncols // 16)) * 16
        buf[r, pl.ds(c, 16)] = jnp.zeros((16,), jnp.float32)
        return 0
    lax.fori_loop(0, nrows * (ncols // 16), body, 0)


def _deg_body(dst_hbm, out_hbm, dst_v, ones_v, acc_sh):
    cid = lax.axis_index("c")
    sid = lax.axis_index("s")
    wid = sid * NC + cid
    pltpu.sync_copy(dst_hbm.at[wid], dst_v)

    base = sid * RPT
    _zero_rows(ones_v, CHUNK, 16)
    for k in range(RPT // CHUNK):
        pltpu.sync_copy(ones_v, acc_sh.at[pl.ds(base + k * CHUNK, CHUNK)])

    def fill(i, _):
        ones_v[i, :] = jnp.ones((16,), jnp.float32)
        return 0
    lax.fori_loop(0, CHUNK, fill, 0)
    plsc.subcore_barrier()

    def body(j, _):
        pltpu.sync_copy(ones_v, acc_sh.at[dst_v.at[j]], add=True)
        return 0
    lax.fori_loop(0, CPW, body, 0)
    plsc.subcore_barrier()
    pltpu.sync_copy(acc_sh.at[pl.ds(base, RPT)],
                    out_hbm.at[cid].at[pl.ds(base, RPT)])


@functools.cache
def _deg_call():
    return pl.kernel(
        _deg_body,
        out_type=jax.ShapeDtypeStruct((NC, NP, 16), jnp.float32),
        mesh=plsc.VectorSubcoreMesh(core_axis_name="c", subcore_axis_name="s"),
        scratch_types=[
            pltpu.VMEM((CPW, CHUNK), jnp.int32),
            pltpu.VMEM((CHUNK, 16), jnp.float32),
            pltpu.VMEM_SHARED((NP, 16), jnp.float32),
        ],
    )


def _mp_body(src_hbm, dst_hbm, t_hbm, out_hbm, src_v, dst_v, rows_v, acc_sh,
             sem):
    cid = lax.axis_index("c")
    sid = lax.axis_index("s")
    wid = sid * NC + cid
    pltpu.sync_copy(src_hbm.at[wid], src_v)
    pltpu.sync_copy(dst_hbm.at[wid], dst_v)

    _zero_rows(rows_v, CHUNK, D)
    base = sid * RPT
    for k in range(RPT // CHUNK):
        pltpu.sync_copy(rows_v, acc_sh.at[pl.ds(base + k * CHUNK, CHUNK)])
    plsc.subcore_barrier()

    def body(j, _):
        pltpu.async_copy(t_hbm.at[src_v.at[j]], rows_v, sem).wait()
        pltpu.sync_copy(rows_v, acc_sh.at[dst_v.at[j]], add=True)
        return 0
    lax.fori_loop(0, CPW, body, 0)
    plsc.subcore_barrier()
    pltpu.sync_copy(acc_sh.at[pl.ds(base, RPT)],
                    out_hbm.at[cid].at[pl.ds(base, RPT)])


@functools.cache
def _mp_call():
    return pl.kernel(
        _mp_body,
        out_type=jax.ShapeDtypeStruct((NC, NP, D), jnp.float32),
        mesh=plsc.VectorSubcoreMesh(core_axis_name="c", subcore_axis_name="s"),
        scratch_types=[
            pltpu.VMEM((CPW, CHUNK), jnp.int32),
            pltpu.VMEM((CPW, CHUNK), jnp.int32),
            pltpu.VMEM((CHUNK, D), jnp.float32),
            pltpu.VMEM_SHARED((NP, D), jnp.float32),
            pltpu.SemaphoreType.DMA,
        ],
    )


def _mm_body(x_ref, w_ref, o_ref):
    o_ref[...] = jnp.dot(x_ref[...], w_ref[...],
                         preferred_element_type=jnp.float32)


_mm_call = pl.pallas_call(
    _mm_body,
    out_shape=jax.ShapeDtypeStruct((NP, D), jnp.float32),
)


def _scale_body(h_ref, degp_ref, t_ref, dinv_ref):
    degp = degp_ref[...]
    deg = degp[0, :, 0:1] + degp[1, :, 0:1] + 1.0
    dinv = lax.rsqrt(deg)
    rid = lax.broadcasted_iota(jnp.int32, (NP, 1), 0)
    dinv = jnp.where(rid < N, dinv, 0.0)
    dinv_b = jnp.broadcast_to(dinv, (NP, D))
    dinv_ref[...] = dinv_b
    t_ref[...] = h_ref[...] * dinv_b


_scale_call = pl.pallas_call(
    _scale_body,
    out_shape=(jax.ShapeDtypeStruct((NP, D), jnp.float32),
               jax.ShapeDtypeStruct((NP, D), jnp.float32)),
)


def _layer_body(s_ref, t_ref, dinv_ref, b_ref, w_ref, o_ref):
    s = s_ref[0] + s_ref[1] + t_ref[...]
    a = jnp.maximum(s * dinv_ref[...] + b_ref[...], 0.0)
    h = jnp.dot(a, w_ref[...], preferred_element_type=jnp.float32)
    o_ref[...] = h * dinv_ref[...]


_layer_call = pl.pallas_call(
    _layer_body,
    out_shape=jax.ShapeDtypeStruct((NP, D), jnp.float32),
)


def _final_body(s_ref, t_ref, dinv_ref, b_ref, batch_ref, wlt_ref, bl_ref,
                o_ref):
    s = s_ref[0] + s_ref[1] + t_ref[...]
    a = s * dinv_ref[...] + b_ref[...]
    ids = lax.broadcasted_iota(jnp.int32, (NG, 1), 0)
    p = (batch_ref[...] == ids).astype(jnp.float32)
    sums = jnp.dot(p, a, preferred_element_type=jnp.float32)
    cnt = jnp.sum(p, axis=1, keepdims=True)
    pooled = sums / jnp.maximum(cnt, 1.0)
    o_ref[...] = jnp.dot(pooled, wlt_ref[...],
                         preferred_element_type=jnp.float32) + bl_ref[...]


_final_call = pl.pallas_call(
    _final_body,
    out_shape=jax.ShapeDtypeStruct((NG, D), jnp.float32),
)


def kernel(x, edge_index, batch, W1, b1, W2, b2, W3, b3, Wl, bl):
    xn = x.reshape(N, D)
    xp = jnp.concatenate([xn, jnp.zeros((NP - N, D), jnp.float32)], axis=0)
    padi = jnp.full((EP - E,), N, jnp.int32)
    srcp = jnp.concatenate([edge_index[0], padi]).reshape(NW, CPW, CHUNK)
    dstp = jnp.concatenate([edge_index[1], padi]).reshape(NW, CPW, CHUNK)
    batchp = jnp.concatenate(
        [batch, jnp.full((NP - N,), NG, jnp.int32)]).reshape(1, NP)

    degp = _deg_call()(dstp)
    h1 = _mm_call(xp, W1)
    t1, dinv_b = _scale_call(h1, degp)

    mp = _mp_call()
    s1 = mp(srcp, dstp, t1)
    t2 = _layer_call(s1, t1, dinv_b, b1.reshape(1, D), W2)
    s2 = mp(srcp, dstp, t2)
    t3 = _layer_call(s2, t2, dinv_b, b2.reshape(1, D), W3)
    s3 = mp(srcp, dstp, t3)

    return _final_call(s3, t3, dinv_b, b3.reshape(1, D), batchp,
                       Wl.T, bl.reshape(1, D))
